# baseline (speedup 1.0000x reference)
import jax
import jax.numpy as jnp
from jax import lax
from jax.experimental import pallas as pl
from jax.experimental.pallas import tpu as pltpu
from jax.experimental.pallas import tpu_sc as plsc

NUM_BUCKETS = 100000
NUM_HEADS = 4
HEAD_DIM = 32
STEP = 9

ROWS = 16384
COLS = 26
N = ROWS * COLS

NC = 2
NS = 16
NW = NC * NS
PER_W = N // NW
CHUNK = 128
NCHUNK = PER_W // CHUNK
CPB = 4
BLK = CPB * CHUNK
NBLK = PER_W // BLK
LANES = 16
K = 4


def _make_body(head, writes_output):
    def body(*args):
        if writes_output:
            idx_hbm, w, out_hbm = args[:3]
        else:
            out_hbm, idx_hbm, w = args[:3]
        raw_v, hidx_v, rows_v = args[3:6]
        gsem = args[6:6 + K]
        wsem = args[6 + K:6 + 2 * K]
        wid = lax.axis_index("s") * NC + lax.axis_index("c")
        wbase = wid * PER_W
        band = pl.ds(head * HEAD_DIM, HEAD_DIM)

        pltpu.sync_copy(idx_hbm.at[pl.ds(wid * NCHUNK, NCHUNK)], raw_v)

        def fire_g(blk, slot):
            for q in range(CPB):
                c = blk * CPB + q
                if head == 0:
                    idx_ref = raw_v.at[c]
                else:
                    for i in range(CHUNK // LANES):
                        sl = pl.ds(i * LANES, LANES)
                        x = raw_v[c, sl]
                        for _ in range(head):
                            x = jnp.where(
                                x >= STEP, x - STEP, x + (NUM_BUCKETS - STEP)
                            )
                        hidx_v[slot, q, sl] = x
                    idx_ref = hidx_v.at[slot, q]
                pltpu.async_copy(
                    w.at[idx_ref],
                    rows_v.at[slot, pl.ds(q * CHUNK, CHUNK), :],
                    gsem[slot],
                )

        def wait_g(slot):
            pltpu.make_async_copy(
                out_hbm.at[pl.ds(0, BLK), band], rows_v.at[slot], gsem[slot]
            ).wait()

        def fire_w(blk, slot):
            base = wbase + blk * BLK
            pltpu.async_copy(
                rows_v.at[slot], out_hbm.at[pl.ds(base, BLK), band], wsem[slot]
            )

        def wait_w(slot):
            pltpu.make_async_copy(
                rows_v.at[slot], out_hbm.at[pl.ds(0, BLK), band], wsem[slot]
            ).wait()

        fire_g(0, 0)
        fire_g(1, 1)
        wait_g(0)
        fire_w(0, 0)
        fire_g(2, 2)
        wait_g(1)
        fire_w(1, 1)
        fire_g(3, 3)

        def main_body(t, carry):
            for b in range(K):
                j = 2 + K * t + b
                s_a = (2 + b) % K
                wait_g(s_a)
                fire_w(j, s_a)
                wait_w(b)
                fire_g(j + 2, b)
            return carry

        lax.fori_loop(0, (NBLK - 6) // K, main_body, 0)

        for j in range(NBLK - 4, NBLK - 2):
            wait_g(j % K)
            fire_w(j, j % K)
            wait_w((j - 2) % K)
            fire_g(j + 2, (j + 2) % K)
        for j in range(NBLK - 2, NBLK):
            wait_g(j % K)
            fire_w(j, j % K)
            wait_w((j - 2) % K)
        wait_w((NBLK - 2) % K)
        wait_w((NBLK - 1) % K)

    return body


def kernel(hashed_value, W0, W1, W2, W3):
    idx_2d = hashed_value.T.reshape(N // CHUNK, CHUNK).astype(jnp.int32)
    mesh = plsc.VectorSubcoreMesh(
        core_axis_name="c", subcore_axis_name="s", num_cores=NC, num_subcores=NS
    )
    params = pltpu.CompilerParams(use_tc_tiling_on_sc=False)
    scratch = (
        [
            pltpu.VMEM((NCHUNK, CHUNK), jnp.int32),
            pltpu.VMEM((K, CPB, CHUNK), jnp.int32),
            pltpu.VMEM((K, BLK, HEAD_DIM), jnp.float32),
        ]
        + [pltpu.SemaphoreType.DMA] * (2 * K)
    )

    k0 = pl.kernel(
        _make_body(0, writes_output=True),
        out_type=jax.ShapeDtypeStruct((N, NUM_HEADS * HEAD_DIM), jnp.float32),
        mesh=mesh,
        scratch_types=scratch,
        compiler_params=params,
    )
    out0 = k0(idx_2d, W0)
    o_ref = jax.new_ref(out0)
    for h, w in ((1, W1), (2, W2), (3, W3)):
        kh = pl.kernel(
            _make_body(h, writes_output=False),
            out_type=(),
            mesh=mesh,
            scratch_types=scratch,
            compiler_params=params,
        )
        kh(o_ref, idx_2d, w)
    out = o_ref[...]
    return out.reshape(COLS, ROWS, NUM_HEADS * HEAD_DIM).transpose(1, 0, 2)


# speedup vs baseline: 1.0523x; 1.0523x over previous
import jax
import jax.numpy as jnp
from jax import lax
from jax.experimental import pallas as pl
from jax.experimental.pallas import tpu as pltpu
from jax.experimental.pallas import tpu_sc as plsc

NUM_BUCKETS = 100000
NUM_HEADS = 4
HEAD_DIM = 32
STEP = 9

ROWS = 16384
COLS = 26
N = ROWS * COLS

NC = 2
NS = 16
NW = NC * NS
PER_W = N // NW
CHUNK = 128
NCHUNK = PER_W // CHUNK
CPB = 4
BLK = CPB * CHUNK
NBLK = PER_W // BLK
LANES = 16
K = 4


def _make_body(heads, writes_output):
    def body(*args):
        if writes_output:
            idx_hbm, w0, w1, out_hbm = args[:4]
        else:
            out_hbm, idx_hbm, w0, w1 = args[:4]
        raw_v, hidx_v, rows_v = args[4:7]
        gsem = args[7:7 + K]
        wsem = args[7 + K:7 + 2 * K]
        wid = lax.axis_index("s") * NC + lax.axis_index("c")
        wbase = wid * PER_W

        pltpu.sync_copy(idx_hbm.at[pl.ds(wid * NCHUNK, NCHUNK)], raw_v)

        for head, w in zip(heads, (w0, w1)):
            _head_pass(
                head, w, out_hbm, raw_v, hidx_v, rows_v, gsem, wsem, wbase
            )

    return body


def _head_pass(head, w, out_hbm, raw_v, hidx_v, rows_v, gsem, wsem, wbase):
        band = pl.ds(head * HEAD_DIM, HEAD_DIM)

        def fire_g(blk, slot):
            for q in range(CPB):
                c = blk * CPB + q
                if head == 0:
                    idx_ref = raw_v.at[c]
                else:
                    for i in range(CHUNK // LANES):
                        sl = pl.ds(i * LANES, LANES)
                        x = raw_v[c, sl]
                        for _ in range(head):
                            x = jnp.where(
                                x >= STEP, x - STEP, x + (NUM_BUCKETS - STEP)
                            )
                        hidx_v[slot, q, sl] = x
                    idx_ref = hidx_v.at[slot, q]
                pltpu.async_copy(
                    w.at[idx_ref],
                    rows_v.at[slot, pl.ds(q * CHUNK, CHUNK), :],
                    gsem[slot],
                )

        def wait_g(slot):
            pltpu.make_async_copy(
                out_hbm.at[pl.ds(0, BLK), band], rows_v.at[slot], gsem[slot]
            ).wait()

        def fire_w(blk, slot):
            base = wbase + blk * BLK
            pltpu.async_copy(
                rows_v.at[slot], out_hbm.at[pl.ds(base, BLK), band], wsem[slot]
            )

        def wait_w(slot):
            pltpu.make_async_copy(
                rows_v.at[slot], out_hbm.at[pl.ds(0, BLK), band], wsem[slot]
            ).wait()

        fire_g(0, 0)
        fire_g(1, 1)
        wait_g(0)
        fire_w(0, 0)
        fire_g(2, 2)
        wait_g(1)
        fire_w(1, 1)
        fire_g(3, 3)

        def main_body(t, carry):
            for b in range(K):
                j = 2 + K * t + b
                s_a = (2 + b) % K
                wait_g(s_a)
                fire_w(j, s_a)
                wait_w(b)
                fire_g(j + 2, b)
            return carry

        lax.fori_loop(0, (NBLK - 6) // K, main_body, 0)

        for j in range(NBLK - 4, NBLK - 2):
            wait_g(j % K)
            fire_w(j, j % K)
            wait_w((j - 2) % K)
            fire_g(j + 2, (j + 2) % K)
        for j in range(NBLK - 2, NBLK):
            wait_g(j % K)
            fire_w(j, j % K)
            wait_w((j - 2) % K)
        wait_w((NBLK - 2) % K)
        wait_w((NBLK - 1) % K)


def kernel(hashed_value, W0, W1, W2, W3):
    idx_2d = hashed_value.T.reshape(N // CHUNK, CHUNK).astype(jnp.int32)
    mesh = plsc.VectorSubcoreMesh(
        core_axis_name="c", subcore_axis_name="s", num_cores=NC, num_subcores=NS
    )
    params = pltpu.CompilerParams(use_tc_tiling_on_sc=False)
    scratch = (
        [
            pltpu.VMEM((NCHUNK, CHUNK), jnp.int32),
            pltpu.VMEM((K, CPB, CHUNK), jnp.int32),
            pltpu.VMEM((K, BLK, HEAD_DIM), jnp.float32),
        ]
        + [pltpu.SemaphoreType.DMA] * (2 * K)
    )

    k01 = pl.kernel(
        _make_body((0, 1), writes_output=True),
        out_type=jax.ShapeDtypeStruct((N, NUM_HEADS * HEAD_DIM), jnp.float32),
        mesh=mesh,
        scratch_types=scratch,
        compiler_params=params,
    )
    out01 = k01(idx_2d, W0, W1)
    o_ref = jax.new_ref(out01)
    k23 = pl.kernel(
        _make_body((2, 3), writes_output=False),
        out_type=(),
        mesh=mesh,
        scratch_types=scratch,
        compiler_params=params,
    )
    k23(o_ref, idx_2d, W2, W3)
    out = o_ref[...]
    return out.reshape(COLS, ROWS, NUM_HEADS * HEAD_DIM).transpose(1, 0, 2)
